# baseline (speedup 1.0000x reference)
import functools

import jax
import jax.numpy as jnp
from jax import lax
from jax.experimental import pallas as pl
from jax.experimental.pallas import tpu as pltpu
from jax.experimental.pallas import tpu_sc as plsc

VOCAB = 100000
EMBED_DIM = 64
BATCH = 4096
SEQ = 26
TOTAL = BATCH * SEQ
PACKED = TOTAL // 2
NUM_WORKERS = 32
PACK_PER_WORKER = PACKED // NUM_WORKERS
CHUNK = 128
STEPS = PACK_PER_WORKER // CHUNK

_MESH = plsc.VectorSubcoreMesh(core_axis_name="c", subcore_axis_name="s")


@functools.partial(
    pl.kernel,
    out_type=jax.ShapeDtypeStruct((PACKED, 2 * EMBED_DIM), jnp.float32),
    mesh=_MESH,
    compiler_params=pltpu.CompilerParams(use_tc_tiling_on_sc=False),
    scratch_types=[
        pltpu.VMEM((PACK_PER_WORKER,), jnp.int32),
        pltpu.VMEM((PACK_PER_WORKER,), jnp.int32),
        pltpu.VMEM((2, CHUNK, EMBED_DIM), jnp.float32),
        pltpu.VMEM((2, CHUNK, EMBED_DIM), jnp.float32),
        pltpu.SemaphoreType.DMA,
        pltpu.SemaphoreType.DMA,
        pltpu.SemaphoreType.DMA,
        pltpu.SemaphoreType.DMA,
    ],
)
def _gather_kernel(table_hbm, eidx_hbm, oidx_hbm, out_hbm,
                   eidx_v, oidx_v, ebufs, obufs, g0, g1, w0, w1):
    wid = lax.axis_index("s") * 2 + lax.axis_index("c")
    pbase = wid * PACK_PER_WORKER

    pltpu.sync_copy(eidx_hbm.at[pl.ds(pbase, PACK_PER_WORKER)], eidx_v)
    pltpu.sync_copy(oidx_hbm.at[pl.ds(pbase, PACK_PER_WORKER)], oidx_v)

    gsems = (g0, g1)
    wsems = (w0, w1)

    def fire(step, b):
        e = pltpu.async_copy(
            table_hbm.at[eidx_v.at[pl.ds(step * CHUNK, CHUNK)]],
            ebufs.at[b], gsems[b])
        o = pltpu.async_copy(
            table_hbm.at[oidx_v.at[pl.ds(step * CHUNK, CHUNK)]],
            obufs.at[b], gsems[b])
        return e, o

    gc = [None, None]
    wc = [None, None]
    gc[0] = fire(0, 0)
    for step in range(STEPS):
        nxt = step + 1
        if nxt < STEPS:
            b = nxt % 2
            if nxt >= 2:
                wc[b][0].wait()
                wc[b][1].wait()
            gc[b] = fire(nxt, b)
        b = step % 2
        gc[b][0].wait()
        gc[b][1].wait()
        rows = pl.ds(pbase + step * CHUNK, CHUNK)
        wc[b] = (
            pltpu.async_copy(ebufs.at[b],
                             out_hbm.at[rows, pl.ds(0, EMBED_DIM)], wsems[b]),
            pltpu.async_copy(obufs.at[b],
                             out_hbm.at[rows, pl.ds(EMBED_DIM, EMBED_DIM)],
                             wsems[b]),
        )
    for b in range(2):
        wc[b][0].wait()
        wc[b][1].wait()


def kernel(indices, table):
    flat = indices.astype(jnp.int32).reshape(PACKED, 2)
    out2d = _gather_kernel(table, flat[:, 0], flat[:, 1])
    return out2d.reshape(BATCH, SEQ, EMBED_DIM)


# speedup vs baseline: 1.1790x; 1.1790x over previous
import functools

import jax
import jax.numpy as jnp
from jax import lax
from jax.experimental import pallas as pl
from jax.experimental.pallas import tpu as pltpu
from jax.experimental.pallas import tpu_sc as plsc

VOCAB = 100000
EMBED_DIM = 64
BATCH = 4096
SEQ = 26
NUM_WORKERS = 32
SLABS_PER_WORKER = BATCH // NUM_WORKERS
GROUP = 16
GROUPS_PER_WORKER = SLABS_PER_WORKER // GROUP

_MESH = plsc.VectorSubcoreMesh(core_axis_name="c", subcore_axis_name="s")


@functools.partial(
    pl.kernel,
    out_type=jax.ShapeDtypeStruct((BATCH, SEQ, EMBED_DIM), jnp.float32),
    mesh=_MESH,
    compiler_params=pltpu.CompilerParams(use_tc_tiling_on_sc=False,
                                         needs_layout_passes=True),
    scratch_types=[
        pltpu.VMEM((SLABS_PER_WORKER, SEQ), jnp.int32),
        pltpu.VMEM((GROUP, SEQ, EMBED_DIM), jnp.float32),
        pltpu.VMEM((GROUP, SEQ, EMBED_DIM), jnp.float32),
        pltpu.SemaphoreType.DMA,
        pltpu.SemaphoreType.DMA,
        pltpu.SemaphoreType.DMA,
        pltpu.SemaphoreType.DMA,
    ],
)
def _gather_kernel(table_hbm, idx_hbm, out_hbm, idx_v, buf0, buf1,
                   g0, g1, w0, w1):
    wid = lax.axis_index("s") * 2 + lax.axis_index("c")
    sbase = wid * SLABS_PER_WORKER

    pltpu.sync_copy(idx_hbm.at[pl.ds(sbase, SLABS_PER_WORKER)], idx_v)

    bufs = (buf0, buf1)
    gsems = (g0, g1)
    wsems = (w0, w1)

    def body(i, _):
        gcopies = [[None] * GROUP, [None] * GROUP]
        for p in range(2):
            g = 2 * i + p

            @pl.when(i > 0)
            def _():
                pltpu.make_async_copy(out_hbm.at[pl.ds(0, GROUP)], bufs[p],
                                      wsems[p]).wait()

            for s in range(GROUP):
                gcopies[p][s] = pltpu.async_copy(
                    table_hbm.at[idx_v.at[g * GROUP + s]],
                    bufs[p].at[s], gsems[p])
        for p in range(2):
            g = 2 * i + p
            for s in range(GROUP):
                gcopies[p][s].wait()
            pltpu.async_copy(bufs[p],
                             out_hbm.at[pl.ds(sbase + g * GROUP, GROUP)],
                             wsems[p])
        return _

    lax.fori_loop(0, GROUPS_PER_WORKER // 2, body, None)

    for p in range(2):
        pltpu.make_async_copy(out_hbm.at[pl.ds(0, GROUP)], bufs[p],
                              wsems[p]).wait()


def kernel(indices, table):
    idx = indices.astype(jnp.int32)
    return _gather_kernel(table, idx)
